# Initial kernel scaffold; baseline (speedup 1.0000x reference)
import jax
import jax.numpy as jnp
from jax.experimental import pallas as pl


def kernel(z_e, emb0, emb1, emb2):
    raise NotImplementedError("write your pallas kernel here")


# speedup vs baseline: 1.3887x; 1.3887x over previous
import jax
import jax.numpy as jnp
from jax.experimental import pallas as pl
from jax.experimental.pallas import tpu as pltpu

_NUM_LEVELS = 3
_NUM_CODES = 512
_EMBED_DIM = 64
_COMMIT_COSTS = (0.25, 0.5, 1.0)
_ROWS = 512


def _rvq_tile(z_ref, z2_ref, e2_ref, e0_ref, e1_ref, e2b_ref,
              zq_ref, idx_ref, stats_ref):
    i = pl.program_id(0)
    z = z_ref[...]
    r = z
    zq_sum = jnp.zeros_like(z)
    embs = (e0_ref[...], e1_ref[...], e2b_ref[...])
    counts = []
    losses = []
    code_iota = jax.lax.broadcasted_iota(jnp.int32, (_ROWS, _NUM_CODES), 1)
    for level in range(_NUM_LEVELS):
        emb = embs[level]
        e2 = e2_ref[level:level + 1, :]
        if level == 0:
            z2 = z2_ref[...]
        else:
            z2 = jnp.sum(r * r, axis=1, keepdims=True)
        prod = jax.lax.dot_general(
            r, emb, (((1,), (1,)), ((), ())),
            preferred_element_type=jnp.float32)
        d = (z2 - 2.0 * prod) + e2
        dmin = jnp.min(d, axis=1, keepdims=True)
        idx = jnp.min(jnp.where(d == dmin, code_iota, _NUM_CODES), axis=1)
        one_hot = (code_iota == idx[:, None]).astype(jnp.float32)
        q = jax.lax.dot_general(
            one_hot, emb, (((1,), (0,)), ((), ())),
            precision=jax.lax.Precision.HIGHEST,
            preferred_element_type=jnp.float32)
        counts.append(jnp.sum(one_hot, axis=0))
        losses.append(jnp.sum((r - q) ** 2))
        idx_ref[level, :] = idx.astype(jnp.int32)
        r = r - q
        zq_sum = zq_sum + q
    zq_ref[...] = z + (zq_sum - z)

    lvl_iota = jax.lax.broadcasted_iota(jnp.int32, (1, _NUM_CODES), 1)
    loss_row = jnp.zeros((1, _NUM_CODES), jnp.float32)
    for level in range(_NUM_LEVELS):
        loss_row = loss_row + jnp.where(lvl_iota == level, losses[level], 0.0)
    new_stats = jnp.concatenate(
        [jnp.stack(counts, axis=0), loss_row], axis=0)

    @pl.when(i == 0)
    def _init():
        stats_ref[...] = new_stats

    @pl.when(i != 0)
    def _accum():
        stats_ref[...] = stats_ref[...] + new_stats


def kernel(z_e, emb0, emb1, emb2):
    shape = z_e.shape
    n = shape[0] * shape[1]
    z_flat = z_e.reshape(n, _EMBED_DIM)
    z2_0 = jnp.sum(z_flat ** 2, axis=1, keepdims=True)
    e2_all = jnp.stack([jnp.sum(emb0 ** 2, axis=1),
                        jnp.sum(emb1 ** 2, axis=1),
                        jnp.sum(emb2 ** 2, axis=1)], axis=0)
    grid = (n // _ROWS,)

    zq_flat, idx_all, stats = pl.pallas_call(
        _rvq_tile,
        grid=grid,
        in_specs=[
            pl.BlockSpec((_ROWS, _EMBED_DIM), lambda i: (i, 0)),
            pl.BlockSpec((_ROWS, 1), lambda i: (i, 0)),
            pl.BlockSpec((_NUM_LEVELS, _NUM_CODES), lambda i: (0, 0)),
            pl.BlockSpec((_NUM_CODES, _EMBED_DIM), lambda i: (0, 0)),
            pl.BlockSpec((_NUM_CODES, _EMBED_DIM), lambda i: (0, 0)),
            pl.BlockSpec((_NUM_CODES, _EMBED_DIM), lambda i: (0, 0)),
        ],
        out_specs=[
            pl.BlockSpec((_ROWS, _EMBED_DIM), lambda i: (i, 0)),
            pl.BlockSpec((_NUM_LEVELS, _ROWS), lambda i: (0, i)),
            pl.BlockSpec((_NUM_LEVELS + 1, _NUM_CODES), lambda i: (0, 0)),
        ],
        out_shape=[
            jax.ShapeDtypeStruct((n, _EMBED_DIM), jnp.float32),
            jax.ShapeDtypeStruct((_NUM_LEVELS, n), jnp.int32),
            jax.ShapeDtypeStruct((_NUM_LEVELS + 1, _NUM_CODES), jnp.float32),
        ],
        compiler_params=pltpu.CompilerParams(
            dimension_semantics=("arbitrary",)),
    )(z_flat, z2_0, e2_all, emb0, emb1, emb2)

    z_q_st = zq_flat.reshape(shape)
    indices = idx_all.reshape(_NUM_LEVELS, shape[0], shape[1])
    counts = stats[:_NUM_LEVELS]
    loss_sums = stats[_NUM_LEVELS, :_NUM_LEVELS]
    denom = jnp.float32(n * _EMBED_DIM)
    total_vq_loss = jnp.sum(
        jnp.asarray(_COMMIT_COSTS, jnp.float32) * (loss_sums / denom))
    avg_probs = counts / jnp.float32(n)
    perps = jnp.exp(-jnp.sum(avg_probs * jnp.log(avg_probs + 1e-10), axis=1))
    return z_q_st, indices, total_vq_loss, perps


# speedup vs baseline: 1.9187x; 1.3816x over previous
import jax
import jax.numpy as jnp
from jax.experimental import pallas as pl
from jax.experimental.pallas import tpu as pltpu

_NUM_LEVELS = 3
_NUM_CODES = 512
_EMBED_DIM = 64
_COMMIT_COSTS = (0.25, 0.5, 1.0)
_ROWS = 512


def _rvq_tile(z_ref, z2_ref, e2_ref, s0_ref, s1_ref, s2_ref,
              zq_ref, idx_ref, stats_ref):
    i = pl.program_id(0)
    z = z_ref[...]
    r = z
    zq_sum = jnp.zeros_like(z)
    splits = (s0_ref, s1_ref, s2_ref)
    counts = []
    losses = []
    code_iota = jax.lax.broadcasted_iota(jnp.int32, (_ROWS, _NUM_CODES), 1)
    for level in range(_NUM_LEVELS):
        s = splits[level]
        b0 = s[0:_NUM_CODES, :]
        b1 = s[_NUM_CODES:2 * _NUM_CODES, :]
        b2 = s[2 * _NUM_CODES:3 * _NUM_CODES, :]
        emb = (b0 + b1) + b2
        e2 = e2_ref[level:level + 1, :]
        if level == 0:
            z2 = z2_ref[...]
        else:
            z2 = jnp.sum(r * r, axis=1, keepdims=True)
        prod = jax.lax.dot_general(
            r, emb, (((1,), (1,)), ((), ())),
            preferred_element_type=jnp.float32)
        d = (z2 - 2.0 * prod) + e2
        dmin = jnp.min(d, axis=1, keepdims=True)
        idx = jnp.min(jnp.where(d == dmin, code_iota, _NUM_CODES), axis=1)
        one_hot = (code_iota == idx[:, None]).astype(jnp.float32)
        def _pick(part):
            return jax.lax.dot_general(
                one_hot, part, (((1,), (0,)), ((), ())),
                preferred_element_type=jnp.float32)
        q = (_pick(b0) + _pick(b1)) + _pick(b2)
        counts.append(jnp.sum(one_hot, axis=0))
        losses.append(jnp.sum((r - q) ** 2))
        idx_ref[level, :] = idx.astype(jnp.int32)
        r = r - q
        zq_sum = zq_sum + q
    zq_ref[...] = z + (zq_sum - z)

    lvl_iota = jax.lax.broadcasted_iota(jnp.int32, (1, _NUM_CODES), 1)
    loss_row = jnp.zeros((1, _NUM_CODES), jnp.float32)
    for level in range(_NUM_LEVELS):
        loss_row = loss_row + jnp.where(lvl_iota == level, losses[level], 0.0)
    new_stats = jnp.concatenate(
        [jnp.stack(counts, axis=0), loss_row], axis=0)

    @pl.when(i == 0)
    def _init():
        stats_ref[...] = new_stats

    @pl.when(i != 0)
    def _accum():
        stats_ref[...] = stats_ref[...] + new_stats


def kernel(z_e, emb0, emb1, emb2):
    shape = z_e.shape
    n = shape[0] * shape[1]
    z_flat = z_e.reshape(n, _EMBED_DIM)
    z2_0 = jnp.sum(z_flat ** 2, axis=1, keepdims=True)
    e2_all = jnp.stack([jnp.sum(emb0 ** 2, axis=1),
                        jnp.sum(emb1 ** 2, axis=1),
                        jnp.sum(emb2 ** 2, axis=1)], axis=0)

    def _split3(e):
        b0 = e.astype(jnp.bfloat16).astype(jnp.float32)
        r1 = e - b0
        b1 = r1.astype(jnp.bfloat16).astype(jnp.float32)
        b2 = r1 - b1
        return jnp.concatenate([b0, b1, b2], axis=0)

    s0, s1, s2 = _split3(emb0), _split3(emb1), _split3(emb2)
    grid = (n // _ROWS,)

    zq_flat, idx_all, stats = pl.pallas_call(
        _rvq_tile,
        grid=grid,
        in_specs=[
            pl.BlockSpec((_ROWS, _EMBED_DIM), lambda i: (i, 0)),
            pl.BlockSpec((_ROWS, 1), lambda i: (i, 0)),
            pl.BlockSpec((_NUM_LEVELS, _NUM_CODES), lambda i: (0, 0)),
            pl.BlockSpec((3 * _NUM_CODES, _EMBED_DIM), lambda i: (0, 0)),
            pl.BlockSpec((3 * _NUM_CODES, _EMBED_DIM), lambda i: (0, 0)),
            pl.BlockSpec((3 * _NUM_CODES, _EMBED_DIM), lambda i: (0, 0)),
        ],
        out_specs=[
            pl.BlockSpec((_ROWS, _EMBED_DIM), lambda i: (i, 0)),
            pl.BlockSpec((_NUM_LEVELS, _ROWS), lambda i: (0, i)),
            pl.BlockSpec((_NUM_LEVELS + 1, _NUM_CODES), lambda i: (0, 0)),
        ],
        out_shape=[
            jax.ShapeDtypeStruct((n, _EMBED_DIM), jnp.float32),
            jax.ShapeDtypeStruct((_NUM_LEVELS, n), jnp.int32),
            jax.ShapeDtypeStruct((_NUM_LEVELS + 1, _NUM_CODES), jnp.float32),
        ],
        compiler_params=pltpu.CompilerParams(
            dimension_semantics=("arbitrary",)),
    )(z_flat, z2_0, e2_all, s0, s1, s2)

    z_q_st = zq_flat.reshape(shape)
    indices = idx_all.reshape(_NUM_LEVELS, shape[0], shape[1])
    counts = stats[:_NUM_LEVELS]
    loss_sums = stats[_NUM_LEVELS, :_NUM_LEVELS]
    denom = jnp.float32(n * _EMBED_DIM)
    total_vq_loss = jnp.sum(
        jnp.asarray(_COMMIT_COSTS, jnp.float32) * (loss_sums / denom))
    avg_probs = counts / jnp.float32(n)
    perps = jnp.exp(-jnp.sum(avg_probs * jnp.log(avg_probs + 1e-10), axis=1))
    return z_q_st, indices, total_vq_loss, perps


# speedup vs baseline: 2.0156x; 1.0505x over previous
import jax
import jax.numpy as jnp
from jax.experimental import pallas as pl
from jax.experimental.pallas import tpu as pltpu

_NUM_LEVELS = 3
_NUM_CODES = 512
_EMBED_DIM = 64
_COMMIT_COSTS = (0.25, 0.5, 1.0)
_ROWS = 512


def _rvq_tile(z_ref, z2_ref, e2_ref, s0_ref, s1_ref, s2_ref,
              zq_ref, idx_ref, stats_ref):
    i = pl.program_id(0)
    z = z_ref[...]
    r = z
    zq_sum = jnp.zeros_like(z)
    splits = (s0_ref, s1_ref, s2_ref)
    counts = []
    losses = []
    code_iota = jax.lax.broadcasted_iota(
        jnp.int32, (_ROWS, _NUM_CODES), 1).astype(jnp.float32)
    for level in range(_NUM_LEVELS):
        s = splits[level]
        b0 = s[0:_NUM_CODES, :]
        b1 = s[_NUM_CODES:2 * _NUM_CODES, :]
        b2 = s[2 * _NUM_CODES:3 * _NUM_CODES, :]
        emb = (b0 + b1) + b2
        e2 = e2_ref[level:level + 1, :]
        if level == 0:
            z2 = z2_ref[...]
        else:
            z2 = jnp.sum(r * r, axis=1, keepdims=True)
        prod = jax.lax.dot_general(
            r, emb, (((1,), (1,)), ((), ())),
            preferred_element_type=jnp.float32)
        d = (z2 - 2.0 * prod) + e2
        dmin = jnp.min(d, axis=1, keepdims=True)
        idxf = jnp.min(jnp.where(d == dmin, code_iota, float(_NUM_CODES)),
                       axis=1, keepdims=True)
        one_hot = (code_iota == idxf).astype(jnp.float32)
        def _pick(part):
            return jax.lax.dot_general(
                one_hot, part, (((1,), (0,)), ((), ())),
                preferred_element_type=jnp.float32)
        q = (_pick(b0) + _pick(b1)) + _pick(b2)
        counts.append(jnp.sum(one_hot, axis=0))
        losses.append(jnp.sum((r - q) ** 2))
        idx_ref[level, :] = idxf[:, 0].astype(jnp.int32)
        r = r - q
        zq_sum = zq_sum + q
    zq_ref[...] = z + (zq_sum - z)

    lvl_iota = jax.lax.broadcasted_iota(jnp.int32, (1, _NUM_CODES), 1)
    loss_row = jnp.zeros((1, _NUM_CODES), jnp.float32)
    for level in range(_NUM_LEVELS):
        loss_row = loss_row + jnp.where(lvl_iota == level, losses[level], 0.0)
    new_stats = jnp.concatenate(
        [jnp.stack(counts, axis=0), loss_row], axis=0)

    @pl.when(i == 0)
    def _init():
        stats_ref[...] = new_stats

    @pl.when(i != 0)
    def _accum():
        stats_ref[...] = stats_ref[...] + new_stats


def kernel(z_e, emb0, emb1, emb2):
    shape = z_e.shape
    n = shape[0] * shape[1]
    z_flat = z_e.reshape(n, _EMBED_DIM)
    z2_0 = jnp.sum(z_flat ** 2, axis=1, keepdims=True)
    e2_all = jnp.stack([jnp.sum(emb0 ** 2, axis=1),
                        jnp.sum(emb1 ** 2, axis=1),
                        jnp.sum(emb2 ** 2, axis=1)], axis=0)

    def _split3(e):
        b0 = e.astype(jnp.bfloat16).astype(jnp.float32)
        r1 = e - b0
        b1 = r1.astype(jnp.bfloat16).astype(jnp.float32)
        b2 = r1 - b1
        return jnp.concatenate([b0, b1, b2], axis=0)

    s0, s1, s2 = _split3(emb0), _split3(emb1), _split3(emb2)
    grid = (n // _ROWS,)

    zq_flat, idx_all, stats = pl.pallas_call(
        _rvq_tile,
        grid=grid,
        in_specs=[
            pl.BlockSpec((_ROWS, _EMBED_DIM), lambda i: (i, 0)),
            pl.BlockSpec((_ROWS, 1), lambda i: (i, 0)),
            pl.BlockSpec((_NUM_LEVELS, _NUM_CODES), lambda i: (0, 0)),
            pl.BlockSpec((3 * _NUM_CODES, _EMBED_DIM), lambda i: (0, 0)),
            pl.BlockSpec((3 * _NUM_CODES, _EMBED_DIM), lambda i: (0, 0)),
            pl.BlockSpec((3 * _NUM_CODES, _EMBED_DIM), lambda i: (0, 0)),
        ],
        out_specs=[
            pl.BlockSpec((_ROWS, _EMBED_DIM), lambda i: (i, 0)),
            pl.BlockSpec((_NUM_LEVELS, _ROWS), lambda i: (0, i)),
            pl.BlockSpec((_NUM_LEVELS + 1, _NUM_CODES), lambda i: (0, 0)),
        ],
        out_shape=[
            jax.ShapeDtypeStruct((n, _EMBED_DIM), jnp.float32),
            jax.ShapeDtypeStruct((_NUM_LEVELS, n), jnp.int32),
            jax.ShapeDtypeStruct((_NUM_LEVELS + 1, _NUM_CODES), jnp.float32),
        ],
        compiler_params=pltpu.CompilerParams(
            dimension_semantics=("arbitrary",)),
    )(z_flat, z2_0, e2_all, s0, s1, s2)

    z_q_st = zq_flat.reshape(shape)
    indices = idx_all.reshape(_NUM_LEVELS, shape[0], shape[1])
    counts = stats[:_NUM_LEVELS]
    loss_sums = stats[_NUM_LEVELS, :_NUM_LEVELS]
    denom = jnp.float32(n * _EMBED_DIM)
    total_vq_loss = jnp.sum(
        jnp.asarray(_COMMIT_COSTS, jnp.float32) * (loss_sums / denom))
    avg_probs = counts / jnp.float32(n)
    perps = jnp.exp(-jnp.sum(avg_probs * jnp.log(avg_probs + 1e-10), axis=1))
    return z_q_st, indices, total_vq_loss, perps
